# Initial kernel scaffold; baseline (speedup 1.0000x reference)
import jax
import jax.numpy as jnp
from jax.experimental import pallas as pl


def kernel(graph_x, graph_edge, W1, b1, W2, b2):
    raise NotImplementedError("write your pallas kernel here")


# speedup vs baseline: 7.2261x; 7.2261x over previous
import functools

import jax
import jax.numpy as jnp
from jax import lax
from jax.experimental import pallas as pl
from jax.experimental.pallas import tpu as pltpu
from jax.experimental.pallas import tpu_sc as plsc

N = 10000
E = 320000
D = 128
HD = D // 2

NC = 2
NS = 16
NW = NC * NS
EPW = E // NW
CHUNK = 128
FIRE = 4
NCHUNK = -(-EPW // (CHUNK * FIRE)) * FIRE
EPW_PAD = NCHUNK * CHUNK
ROWS_Z = 632
ACC_R = NS * ROWS_Z
DW = 16

_mesh = plsc.VectorSubcoreMesh(core_axis_name="c", subcore_axis_name="s",
                               num_cores=NC)
_sc_params = pltpu.CompilerParams(use_tc_tiling_on_sc=False)


@functools.partial(
    pl.kernel,
    out_type=jax.ShapeDtypeStruct((NC, ACC_R, DW), jnp.float32),
    mesh=_mesh,
    compiler_params=_sc_params,
    scratch_types=[
        pltpu.VMEM((NCHUNK, CHUNK), jnp.int32),
        pltpu.VMEM((CHUNK, DW), jnp.float32),
        pltpu.VMEM_SHARED((ACC_R, DW), jnp.float32),
        pltpu.SemaphoreType.DMA,
    ],
)
def _sc_degree(dstw_hbm, out_hbm, dst_v, ones_v, acc, sem):
    c = lax.axis_index("c")
    s = lax.axis_index("s")
    wid = s * NC + c
    pltpu.sync_copy(dstw_hbm.at[wid], dst_v)

    def zfill(r, carry):
        ones_v[r, :] = jnp.zeros((16,), jnp.float32)
        return carry

    lax.fori_loop(0, CHUNK, zfill, 0, unroll=False)
    base = s * ROWS_Z
    for off, ln in ((0, CHUNK), (CHUNK, CHUNK), (2 * CHUNK, CHUNK),
                    (3 * CHUNK, CHUNK), (4 * CHUNK, ROWS_Z - 4 * CHUNK)):
        pltpu.sync_copy(ones_v.at[pl.ds(0, ln)], acc.at[pl.ds(base + off, ln)])

    onerow = jnp.where(lax.iota(jnp.int32, 16) < 1, 1.0, 0.0)

    def fill(r, carry):
        ones_v[r, :] = onerow
        return carry

    lax.fori_loop(0, CHUNK, fill, 0, unroll=False)
    plsc.subcore_barrier()

    def step(i, carry):
        cps = []
        for b in range(FIRE):
            j = i * FIRE + b
            cps.append(pltpu.async_copy(ones_v, acc.at[dst_v.at[j]], sem,
                                        add=True))
        for cp in cps:
            cp.wait()
        return carry

    lax.fori_loop(0, NCHUNK // FIRE, step, 0, unroll=False)
    plsc.subcore_barrier()
    pltpu.sync_copy(acc.at[pl.ds(s * ROWS_Z, ROWS_Z)],
                    out_hbm.at[c, pl.ds(s * ROWS_Z, ROWS_Z)])


@functools.partial(
    pl.kernel,
    out_type=jax.ShapeDtypeStruct((2 * NC, ACC_R, HD), jnp.float32),
    mesh=_mesh,
    compiler_params=_sc_params,
    scratch_types=[
        pltpu.VMEM((NCHUNK, CHUNK), jnp.int32),
        pltpu.VMEM((NCHUNK, CHUNK), jnp.int32),
        pltpu.VMEM((FIRE * CHUNK, HD), jnp.float32),
        pltpu.VMEM_SHARED((ACC_R, HD), jnp.float32),
        pltpu.SemaphoreType.DMA,
        pltpu.SemaphoreType.DMA,
    ],
)
def _sc_scatter(p0_hbm, p1_hbm, srcw_hbm, dstw_hbm, out_hbm,
                src_v, dst_v, buf, acc, gsem, ssem):
    c = lax.axis_index("c")
    s = lax.axis_index("s")
    wid = s * NC + c
    pltpu.sync_copy(srcw_hbm.at[wid], src_v)
    pltpu.sync_copy(dstw_hbm.at[wid], dst_v)
    base = s * ROWS_Z

    for t, p_hbm in enumerate((p0_hbm, p1_hbm)):
        def zfill(r, carry):
            for k in range(HD // 16):
                buf[r, pl.ds(k * 16, 16)] = jnp.zeros((16,), jnp.float32)
            return carry

        lax.fori_loop(0, CHUNK, zfill, 0, unroll=False)
        for off, ln in ((0, CHUNK), (CHUNK, CHUNK), (2 * CHUNK, CHUNK),
                        (3 * CHUNK, CHUNK), (4 * CHUNK, ROWS_Z - 4 * CHUNK)):
            pltpu.sync_copy(buf.at[pl.ds(0, ln)],
                            acc.at[pl.ds(base + off, ln)])
        plsc.subcore_barrier()

        def step(i, carry):
            gs = []
            for b in range(FIRE):
                j = i * FIRE + b
                gs.append(pltpu.async_copy(p_hbm.at[src_v.at[j]],
                                           buf.at[pl.ds(b * CHUNK, CHUNK)],
                                           gsem))
            for cp in gs:
                cp.wait()
            ss = []
            for b in range(FIRE):
                j = i * FIRE + b
                ss.append(pltpu.async_copy(buf.at[pl.ds(b * CHUNK, CHUNK)],
                                           acc.at[dst_v.at[j]], ssem,
                                           add=True))
            for cp in ss:
                cp.wait()
            return carry

        lax.fori_loop(0, NCHUNK // FIRE, step, 0, unroll=False)
        plsc.subcore_barrier()
        pltpu.sync_copy(acc.at[pl.ds(base, ROWS_Z)],
                        out_hbm.at[t * NC + c, pl.ds(base, ROWS_Z)])


_BLK = 400


def _tc_prep_body(degp_ref, x_ref, wlo_ref, whi_ref, p0_ref, p1_ref, dis_ref):
    deg = sum(degp_ref[c, :, 0:1] for c in range(NC)) + 1.0
    dis = lax.rsqrt(deg)
    dis_ref[...] = dis
    x = x_ref[...]
    p0_ref[...] = jnp.dot(x, wlo_ref[...],
                          preferred_element_type=jnp.float32) * dis
    p1_ref[...] = jnp.dot(x, whi_ref[...],
                          preferred_element_type=jnp.float32) * dis


def _tc_prep(degp, x, wlo, whi):
    return pl.pallas_call(
        _tc_prep_body,
        grid=(N // _BLK,),
        in_specs=[
            pl.BlockSpec((NC, _BLK, DW), lambda i: (0, i, 0)),
            pl.BlockSpec((_BLK, D), lambda i: (i, 0)),
            pl.BlockSpec((D, HD), lambda i: (0, 0)),
            pl.BlockSpec((D, HD), lambda i: (0, 0)),
        ],
        out_specs=[
            pl.BlockSpec((_BLK, HD), lambda i: (i, 0)),
            pl.BlockSpec((_BLK, HD), lambda i: (i, 0)),
            pl.BlockSpec((_BLK, 1), lambda i: (i, 0)),
        ],
        out_shape=[
            jax.ShapeDtypeStruct((N, HD), jnp.float32),
            jax.ShapeDtypeStruct((N, HD), jnp.float32),
            jax.ShapeDtypeStruct((N, 1), jnp.float32),
        ],
    )(degp, x, wlo, whi)


def _tc_mid_body(s_ref, p0_ref, p1_ref, dis_ref, b_ref, flag_ref, xres_ref,
                 wlo_ref, whi_ref, h_ref, pn0_ref, pn1_ref):
    dis = dis_ref[...]
    s_lo = s_ref[0] + s_ref[1] + p0_ref[...]
    s_hi = s_ref[2] + s_ref[3] + p1_ref[...]
    pre = dis * jnp.concatenate([s_lo, s_hi], axis=1) + b_ref[0:1, :]
    act = jnp.where(flag_ref[0:1, :] > 0.0, jnp.maximum(pre, 0.0), pre)
    h = act + xres_ref[...]
    h_ref[...] = h
    pn0_ref[...] = jnp.dot(h, wlo_ref[...],
                           preferred_element_type=jnp.float32) * dis
    pn1_ref[...] = jnp.dot(h, whi_ref[...],
                           preferred_element_type=jnp.float32) * dis


def _tc_mid(s, p0, p1, dis, b8, flag8, xres, wlo, whi):
    return pl.pallas_call(
        _tc_mid_body,
        grid=(N // _BLK,),
        in_specs=[
            pl.BlockSpec((2 * NC, _BLK, HD), lambda i: (0, i, 0)),
            pl.BlockSpec((_BLK, HD), lambda i: (i, 0)),
            pl.BlockSpec((_BLK, HD), lambda i: (i, 0)),
            pl.BlockSpec((_BLK, 1), lambda i: (i, 0)),
            pl.BlockSpec((8, D), lambda i: (0, 0)),
            pl.BlockSpec((8, D), lambda i: (0, 0)),
            pl.BlockSpec((_BLK, D), lambda i: (i, 0)),
            pl.BlockSpec((D, HD), lambda i: (0, 0)),
            pl.BlockSpec((D, HD), lambda i: (0, 0)),
        ],
        out_specs=[
            pl.BlockSpec((_BLK, D), lambda i: (i, 0)),
            pl.BlockSpec((_BLK, HD), lambda i: (i, 0)),
            pl.BlockSpec((_BLK, HD), lambda i: (i, 0)),
        ],
        out_shape=[
            jax.ShapeDtypeStruct((N, D), jnp.float32),
            jax.ShapeDtypeStruct((N, HD), jnp.float32),
            jax.ShapeDtypeStruct((N, HD), jnp.float32),
        ],
    )(s, p0, p1, dis, b8, flag8, xres, wlo, whi)


def kernel(graph_x, graph_edge, W1, b1, W2, b2):
    src = graph_edge[0].astype(jnp.int32)
    dst = graph_edge[1].astype(jnp.int32)

    pad = EPW_PAD - EPW
    srcw = jnp.concatenate(
        [src.reshape(NW, EPW), jnp.zeros((NW, pad), jnp.int32)], axis=1
    ).reshape(NW, NCHUNK, CHUNK)
    dstw = jnp.concatenate(
        [dst.reshape(NW, EPW), jnp.full((NW, pad), N, jnp.int32)], axis=1
    ).reshape(NW, NCHUNK, CHUNK)

    b1_8 = jnp.tile(b1.reshape(1, D), (8, 1))
    b2_8 = jnp.tile(b2.reshape(1, D), (8, 1))
    b_stack = jnp.stack([b1_8, b2_8, b2_8])
    flag_stack = jnp.stack([jnp.full((8, D), f, jnp.float32)
                            for f in (1.0, 1.0, 0.0)])

    w1lo, w1hi = W1[:, :HD], W1[:, HD:]
    w2lo, w2hi = W2[:, :HD], W2[:, HD:]

    degp = _sc_degree(dstw)
    p0, p1, dis = _tc_prep(degp, graph_x, w1lo, w1hi)

    def layer(carry, xs):
        xres, q0, q1 = carry
        b8, flag8 = xs
        s = _sc_scatter(q0, q1, srcw, dstw)
        h_new, pn0, pn1 = _tc_mid(s, q0, q1, dis, b8, flag8, xres, w2lo, w2hi)
        return (h_new, pn0, pn1), 0.0

    (out, _, _), _ = lax.scan(layer, (graph_x, p0, p1),
                              (b_stack, flag_stack))
    return out
